# baseline (speedup 1.0000x reference)
import math
import jax
import jax.numpy as jnp
from jax import lax
from jax.experimental import pallas as pl
from jax.experimental.pallas import tpu as pltpu
from jax.experimental.pallas import tpu_sc as plsc

_B = 16
_TOTAL_STEPS = 8
_CURRENT_STEP = 4
_LP = 512
_LA = 1024
_VOCAB = 1024
_D = 1024
_LANES = 16
_CHUNK = 32


def _embed_kernel(prompts, tokens, table, out,
                  ipx, itx, acc_v, db0, db1, sem_a, sem_b0, sem_b1):
    nc = 2
    wid = lax.axis_index("s") * nc + lax.axis_index("c")
    b = wid // 2
    half = wid % 2
    scale = math.sqrt(float(_D))
    dbufs = (db0, db1)
    sems = (sem_b0, sem_b1)

    def accum(buf):
        def body(r, _):
            for j in range(0, _D, _LANES):
                sl = pl.ds(j, _LANES)
                plsc.addupdate(acc_v.at[r, sl], buf[r, sl])
            return 0
        lax.fori_loop(0, _CHUNK, body, 0)

    def accum_scale(buf):
        def body(r, _):
            for j in range(0, _D, _LANES):
                sl = pl.ds(j, _LANES)
                acc_v[r, sl] = (acc_v[r, sl] + buf[r, sl]) * scale
            return 0
        lax.fori_loop(0, _CHUNK, body, 0)

    def do_section(idx_hbm, idx_v, num_steps, t0, nrows, out_t0):

        pltpu.sync_copy(idx_hbm.at[b, :, pl.ds(t0, nrows)], idx_v)
        for i in range(1, num_steps):
            off = jnp.full((_LANES,), i * _VOCAB, dtype=jnp.int32)
            for j in range(0, nrows, _LANES):
                sl = pl.ds(j, _LANES)
                idx_v[i, sl] = idx_v[i, sl] + off

        def chunk_body(ci, _):
            c0 = pl.multiple_of(ci * _CHUNK, _CHUNK)
            csl = pl.ds(c0, _CHUNK)
            cap0 = pltpu.async_copy(table.at[idx_v.at[0, csl]], acc_v, sem_a)
            caps = {1: pltpu.async_copy(table.at[idx_v.at[1, csl]], dbufs[0], sems[0])}
            cap0.wait()
            for i in range(2, num_steps):
                s = (i - 1) % 2
                caps[i - 1].wait()
                caps[i] = pltpu.async_copy(table.at[idx_v.at[i, csl]], dbufs[s], sems[s])
                accum(dbufs[(i - 2) % 2])
            caps[num_steps - 1].wait()
            accum_scale(dbufs[(num_steps - 2) % 2])
            pltpu.sync_copy(acc_v, out.at[b, pl.ds(out_t0 + t0 + c0, _CHUNK)])
            return 0

        lax.fori_loop(0, nrows // _CHUNK, chunk_body, 0)

    do_section(prompts, ipx, _TOTAL_STEPS, half * (_LP // 2), _LP // 2, 0)
    do_section(tokens, itx, _CURRENT_STEP, half * (_LA // 2), _LA // 2, _LP)


def kernel(acoustic_prompts, acoustic_tokens, a_embeds):
    b, total_steps, lp = acoustic_prompts.shape
    current_step = acoustic_tokens.shape[1]
    la = acoustic_tokens.shape[2]
    d = a_embeds.shape[-1]

    prompts = acoustic_prompts.astype(jnp.int32)
    tokens = acoustic_tokens.astype(jnp.int32)
    table = a_embeds.reshape(total_steps * _VOCAB, d)

    mesh = plsc.VectorSubcoreMesh(
        core_axis_name="c", subcore_axis_name="s", num_cores=2, num_subcores=16
    )
    embeds = pl.kernel(
        _embed_kernel,
        out_type=jax.ShapeDtypeStruct((b, lp + la, d), jnp.float32),
        mesh=mesh,
        scratch_types=[
            pltpu.VMEM((_TOTAL_STEPS, _LP // 2), jnp.int32),
            pltpu.VMEM((_CURRENT_STEP, _LA // 2), jnp.int32),
            pltpu.VMEM((_CHUNK, _D), jnp.float32),
            pltpu.VMEM((_CHUNK, _D), jnp.float32),
            pltpu.VMEM((_CHUNK, _D), jnp.float32),
            pltpu.SemaphoreType.DMA,
            pltpu.SemaphoreType.DMA,
            pltpu.SemaphoreType.DMA,
        ],
    )(prompts, tokens, table)

    seq_len = lp + la
    seq_lens = jnp.full((b,), seq_len, dtype=jnp.int32)
    padding_mask = jnp.arange(seq_len)[None, :] >= seq_lens[:, None]
    return embeds, padding_mask, current_step - 1


# speedup vs baseline: 1.3911x; 1.3911x over previous
import math
import jax
import jax.numpy as jnp
from jax import lax
from jax.experimental import pallas as pl
from jax.experimental.pallas import tpu as pltpu
from jax.experimental.pallas import tpu_sc as plsc

_B = 16
_TOTAL_STEPS = 8
_CURRENT_STEP = 4
_LP = 512
_LA = 1024
_VOCAB = 1024
_D = 1024
_LANES = 16
_CHUNK = 32


def _embed_kernel(prompts, tokens, table, out,
                  ipx, itx, ix0, ib0, ib1, acc_v, db0, db1,
                  sem_a, sem_b0, sem_b1):
    nc = 2
    wid = lax.axis_index("s") * nc + lax.axis_index("c")
    b = wid // 2
    half = wid % 2
    scale = math.sqrt(float(_D))
    dbufs = (db0, db1)
    ibufs = (ib0, ib1)
    sems = (sem_b0, sem_b1)

    def stage(dst, idx_v, i, c0):
        for j in range(0, _CHUNK, _LANES):
            dst[pl.ds(j, _LANES)] = idx_v[i, pl.ds(c0 + j, _LANES)]

    def accum(buf):
        def body(r, _):
            for j in range(0, _D, _LANES):
                sl = pl.ds(j, _LANES)
                plsc.addupdate(acc_v.at[r, sl], buf[r, sl])
            return 0
        lax.fori_loop(0, _CHUNK, body, 0)

    def accum_scale(buf):
        def body(r, _):
            for j in range(0, _D, _LANES):
                sl = pl.ds(j, _LANES)
                acc_v[r, sl] = (acc_v[r, sl] + buf[r, sl]) * scale
            return 0
        lax.fori_loop(0, _CHUNK, body, 0)

    def do_section(idx_hbm, idx_v, num_steps, t0, nrows, out_t0):

        pltpu.sync_copy(idx_hbm.at[b, :, pl.ds(t0, nrows)], idx_v)
        for i in range(1, num_steps):
            off = jnp.full((_LANES,), i * _VOCAB, dtype=jnp.int32)
            for j in range(0, nrows, _LANES):
                sl = pl.ds(j, _LANES)
                idx_v[i, sl] = idx_v[i, sl] + off

        def chunk_body(ci, _):
            c0 = pl.multiple_of(ci * _CHUNK, _CHUNK)
            stage(ix0, idx_v, 0, c0)
            cap0 = pltpu.async_copy(table.at[ix0], acc_v, sem_a)
            stage(ibufs[0], idx_v, 1, c0)
            caps = {1: pltpu.async_copy(table.at[ibufs[0]], dbufs[0], sems[0])}
            cap0.wait()
            for i in range(2, num_steps):
                s = (i - 1) % 2
                caps[i - 1].wait()
                stage(ibufs[s], idx_v, i, c0)
                caps[i] = pltpu.async_copy(table.at[ibufs[s]], dbufs[s], sems[s])
                accum(dbufs[(i - 2) % 2])
            caps[num_steps - 1].wait()
            accum_scale(dbufs[(num_steps - 2) % 2])
            pltpu.sync_copy(acc_v, out.at[b, pl.ds(out_t0 + t0 + c0, _CHUNK)])
            return 0

        lax.fori_loop(0, nrows // _CHUNK, chunk_body, 0)

    do_section(prompts, ipx, _TOTAL_STEPS, half * (_LP // 2), _LP // 2, 0)
    do_section(tokens, itx, _CURRENT_STEP, half * (_LA // 2), _LA // 2, _LP)


def kernel(acoustic_prompts, acoustic_tokens, a_embeds):
    b, total_steps, lp = acoustic_prompts.shape
    current_step = acoustic_tokens.shape[1]
    la = acoustic_tokens.shape[2]
    d = a_embeds.shape[-1]

    prompts = acoustic_prompts.astype(jnp.int32)
    tokens = acoustic_tokens.astype(jnp.int32)
    table = a_embeds.reshape(total_steps * _VOCAB, d)

    mesh = plsc.VectorSubcoreMesh(
        core_axis_name="c", subcore_axis_name="s", num_cores=2, num_subcores=16
    )
    embeds = pl.kernel(
        _embed_kernel,
        out_type=jax.ShapeDtypeStruct((b, lp + la, d), jnp.float32),
        mesh=mesh,
        scratch_types=[
            pltpu.VMEM((_TOTAL_STEPS, _LP // 2), jnp.int32),
            pltpu.VMEM((_CURRENT_STEP, _LA // 2), jnp.int32),
            pltpu.VMEM((_CHUNK,), jnp.int32),
            pltpu.VMEM((_CHUNK,), jnp.int32),
            pltpu.VMEM((_CHUNK,), jnp.int32),
            pltpu.VMEM((_CHUNK, _D), jnp.float32),
            pltpu.VMEM((_CHUNK, _D), jnp.float32),
            pltpu.VMEM((_CHUNK, _D), jnp.float32),
            pltpu.SemaphoreType.DMA,
            pltpu.SemaphoreType.DMA,
            pltpu.SemaphoreType.DMA,
        ],
    )(prompts, tokens, table)

    seq_len = lp + la
    seq_lens = jnp.full((b,), seq_len, dtype=jnp.int32)
    padding_mask = jnp.arange(seq_len)[None, :] >= seq_lens[:, None]
    return embeds, padding_mask, current_step - 1
